# Initial kernel scaffold; baseline (speedup 1.0000x reference)
import jax
import jax.numpy as jnp
from jax.experimental import pallas as pl


def kernel(x, w1, b1, gamma1, beta1, w2, b2, gamma2, beta2):
    raise NotImplementedError("write your pallas kernel here")


# speedup vs baseline: 1.6161x; 1.6161x over previous
import functools

import jax
import jax.numpy as jnp
from jax.experimental import pallas as pl
from jax.experimental.pallas import tpu as pltpu


def _conv_body(x_ref, w_ref, b_ref, scale_ref, shift_ref,
               y_ref, ps_ref, sq_ref, *, apply_act, W):
    _, Cin, HW = x_ref.shape

    x = x_ref[0]
    if apply_act:
        z = x.astype(jnp.float32) * scale_ref[...] + shift_ref[...]
        x = jnp.where(z >= 0, z, 0.01 * z)
    xb = x.astype(jnp.bfloat16)

    zpad = jnp.zeros((Cin, W + 1), jnp.bfloat16)
    xp = jnp.concatenate([zpad, xb, zpad], axis=1)

    col = jax.lax.broadcasted_iota(jnp.int32, (1, HW), 1) % W
    not_first = col != 0
    not_last = col != W - 1

    taps = []
    for kh in range(3):
        for kw in range(3):
            t = jax.lax.slice_in_dim(xp, kh * W + kw, kh * W + kw + HW, axis=1)
            if kw == 0:
                t = jnp.where(not_first, t, jnp.bfloat16(0))
            elif kw == 2:
                t = jnp.where(not_last, t, jnp.bfloat16(0))
            taps.append(t)
    patch = jnp.concatenate(taps, axis=0)

    acc = jnp.dot(w_ref[...], patch,
                  preferred_element_type=jnp.float32)
    acc = acc + b_ref[...]

    ps_ref[0] = jnp.sum(acc, axis=1, keepdims=True)
    sq_ref[0] = jnp.sum(acc * acc, axis=1, keepdims=True)
    y_ref[0] = acc.astype(y_ref.dtype)


def _conv3x3(x2d, wt, b, scale, shift, *, apply_act, W):
    N, Cin, HW = x2d.shape
    Cout = wt.shape[0]
    body = functools.partial(_conv_body, apply_act=apply_act, W=W)
    return pl.pallas_call(
        body,
        out_shape=(
            jax.ShapeDtypeStruct((N, Cout, HW), jnp.bfloat16),
            jax.ShapeDtypeStruct((N, Cout, 1), jnp.float32),
            jax.ShapeDtypeStruct((N, Cout, 1), jnp.float32),
        ),
        grid_spec=pltpu.PrefetchScalarGridSpec(
            num_scalar_prefetch=0,
            grid=(N,),
            in_specs=[
                pl.BlockSpec((1, Cin, HW), lambda n: (n, 0, 0)),
                pl.BlockSpec((Cout, 9 * Cin), lambda n: (0, 0)),
                pl.BlockSpec((Cout, 1), lambda n: (0, 0)),
                pl.BlockSpec((Cin, 1), lambda n: (0, 0)),
                pl.BlockSpec((Cin, 1), lambda n: (0, 0)),
            ],
            out_specs=[
                pl.BlockSpec((1, Cout, HW), lambda n: (n, 0, 0)),
                pl.BlockSpec((1, Cout, 1), lambda n: (n, 0, 0)),
                pl.BlockSpec((1, Cout, 1), lambda n: (n, 0, 0)),
            ],
        ),
        compiler_params=pltpu.CompilerParams(
            dimension_semantics=("parallel",)),
    )(x2d, wt, b, scale, shift)


def _affine_body(y_ref, scale_ref, shift_ref, o_ref):
    z = y_ref[...].astype(jnp.float32) * scale_ref[...] + shift_ref[...]
    o_ref[...] = jnp.where(z >= 0, z, 0.01 * z)


def _affine_lrelu(y, scale, shift, *, nb):
    N, C, HW = y.shape
    return pl.pallas_call(
        _affine_body,
        out_shape=jax.ShapeDtypeStruct((N, C, HW), jnp.float32),
        grid_spec=pltpu.PrefetchScalarGridSpec(
            num_scalar_prefetch=0,
            grid=(N // nb,),
            in_specs=[
                pl.BlockSpec((nb, C, HW), lambda i: (i, 0, 0)),
                pl.BlockSpec((C, 1), lambda i: (0, 0)),
                pl.BlockSpec((C, 1), lambda i: (0, 0)),
            ],
            out_specs=pl.BlockSpec((nb, C, HW), lambda i: (i, 0, 0)),
        ),
        compiler_params=pltpu.CompilerParams(
            dimension_semantics=("parallel",)),
    )(y, scale, shift)


def _bn_fold(ps, sq, gamma, beta, count, eps=1e-5):
    s = jnp.sum(ps[:, :, 0], axis=0)
    q = jnp.sum(sq[:, :, 0], axis=0)
    mean = s / count
    var = jnp.maximum(q / count - mean * mean, 0.0)
    scale = gamma[0] / jnp.sqrt(var + eps)
    shift = beta[0] - mean * scale
    return scale.reshape(-1, 1), shift.reshape(-1, 1)


def kernel(x, w1, b1, gamma1, beta1, w2, b2, gamma2, beta2):
    N, Cin, H, W = x.shape
    HW = H * W
    C1 = w1.shape[-1]
    C2 = w2.shape[-1]

    x2 = x.reshape(N, Cin, HW)
    w1t = w1.reshape(9 * Cin, C1).T.astype(jnp.bfloat16)
    w2t = w2.reshape(9 * C1, C2).T.astype(jnp.bfloat16)
    b1c = b1.reshape(C1, 1).astype(jnp.float32)
    b2c = b2.reshape(C2, 1).astype(jnp.float32)
    ones = jnp.ones((Cin, 1), jnp.float32)
    zeros = jnp.zeros((Cin, 1), jnp.float32)

    y1, ps1, sq1 = _conv3x3(x2, w1t, b1c, ones, zeros, apply_act=False, W=W)
    scale1, shift1 = _bn_fold(ps1, sq1, gamma1, beta1, N * HW)

    y2, ps2, sq2 = _conv3x3(y1, w2t, b2c, scale1, shift1, apply_act=True, W=W)
    scale2, shift2 = _bn_fold(ps2, sq2, gamma2, beta2, N * HW)

    out = _affine_lrelu(y2, scale2, shift2, nb=4)
    return out.reshape(N, C2, H, W)
